# Initial kernel scaffold; baseline (speedup 1.0000x reference)
import jax
import jax.numpy as jnp
from jax.experimental import pallas as pl


def kernel(adj_input, r_index, r_val, ent_matrix, rel_matrix, ent_emb, rel_emb, e_gate_kernel, e_proxy, e_bias, e_attn_0, e_attn_1, r_gate_kernel, r_proxy, r_bias, r_attn_0, r_attn_1):
    raise NotImplementedError("write your pallas kernel here")


# speedup vs baseline: 3.0525x; 3.0525x over previous
import functools

import jax
import jax.numpy as jnp
from jax import lax
from jax.experimental import pallas as pl

N_NODE = 10000
N_REL = 1000
N_E = 320000
N_T = 320000
D = 128
F3 = D * 3


def _l2n(x, axis=-1, eps=1e-12):
    n = jnp.sqrt(jnp.sum(x * x, axis=axis, keepdims=True))
    return x / jnp.maximum(n, eps)


def _seg_mean(idx, emb):
    row, col = idx[:, 0], idx[:, 1]
    cnt = jax.ops.segment_sum(jnp.ones((idx.shape[0],), jnp.float32), row,
                              num_segments=N_NODE)
    s = jax.ops.segment_sum(emb[col], row, num_segments=N_NODE)
    return s / jnp.maximum(cnt, 1.0)[:, None]


def _final_block(out_e_ref, out_r_ref, ep_ref, rp_ref, eg_ref, rg_ref,
                 eb_ref, rb_ref, o_ref):
    def one(x, p, gk, b):
        xn = x * lax.rsqrt(jnp.maximum(jnp.sum(x * x, axis=1, keepdims=True),
                                       1e-24))
        pn = p * lax.rsqrt(jnp.maximum(jnp.sum(p * p, axis=1, keepdims=True),
                                       1e-24))
        logits = jnp.dot(xn, pn.T, preferred_element_type=jnp.float32)
        m = jnp.max(logits, axis=1, keepdims=True)
        e = jnp.exp(logits - m)
        att = e / jnp.sum(e, axis=1, keepdims=True)
        pf = x - jnp.dot(att, p, preferred_element_type=jnp.float32)
        g = jnp.dot(pf, gk, preferred_element_type=jnp.float32) + b
        g = 1.0 / (1.0 + jnp.exp(-g))
        return g * x + (1.0 - g) * pf

    o_ref[:, :F3] = one(out_e_ref[...], ep_ref[...], eg_ref[...], eb_ref[...])
    o_ref[:, F3:] = one(out_r_ref[...], rp_ref[...], rg_ref[...], rb_ref[...])


_BLK = 400


def _final_stage(out_e, out_r, e_proxy, r_proxy, e_gate, r_gate, e_bias,
                 r_bias):
    grid = (N_NODE // _BLK,)
    full = lambda s: pl.BlockSpec(s, lambda i: (0, 0))
    return pl.pallas_call(
        _final_block,
        grid=grid,
        in_specs=[
            pl.BlockSpec((_BLK, F3), lambda i: (i, 0)),
            pl.BlockSpec((_BLK, F3), lambda i: (i, 0)),
            full((64, F3)), full((64, F3)),
            full((F3, F3)), full((F3, F3)),
            full((1, F3)), full((1, F3)),
        ],
        out_specs=pl.BlockSpec((_BLK, 2 * F3), lambda i: (i, 0)),
        out_shape=jax.ShapeDtypeStruct((N_NODE, 2 * F3), jnp.float32),
    )(out_e, out_r, e_proxy, r_proxy, e_gate, r_gate, e_bias, r_bias)


def kernel(adj_input, r_index, r_val, ent_matrix, rel_matrix, ent_emb,
           rel_emb, e_gate_kernel, e_proxy, e_bias, e_attn_0, e_attn_1,
           r_gate_kernel, r_proxy, r_bias, r_attn_0, r_attn_1):
    feat_e = jnp.tanh(_seg_mean(ent_matrix, ent_emb))
    feat_r = jnp.tanh(_seg_mean(rel_matrix, rel_emb))

    h = jax.ops.segment_sum(r_val[:, None] * rel_emb[r_index[:, 1]],
                            r_index[:, 0], num_segments=N_T)
    h = _l2n(h, axis=1)
    ak = jnp.concatenate([e_attn_0, e_attn_1, r_attn_0, r_attn_1], axis=1)
    att1 = jnp.dot(h, ak)
    w = jnp.exp(att1)

    src, dst = adj_input[:, 0], adj_input[:, 1]

    def layer(feat, wcol):
        neigh = feat[dst]
        refl = neigh - 2.0 * jnp.sum(neigh * h, axis=1, keepdims=True) * h
        num = jax.ops.segment_sum(refl * wcol[:, None], src,
                                  num_segments=N_NODE)
        den = jax.ops.segment_sum(wcol, src, num_segments=N_NODE)
        return jnp.tanh(num / jnp.maximum(den, 1e-30)[:, None])

    outs_e = [feat_e]
    outs_r = [feat_r]
    for l in range(2):
        outs_e.append(layer(outs_e[-1], w[:, l]))
        outs_r.append(layer(outs_r[-1], w[:, 2 + l]))

    out_e = jnp.concatenate(outs_e, axis=1)
    out_r = jnp.concatenate(outs_r, axis=1)

    return _final_stage(out_e, out_r, e_proxy, r_proxy, e_gate_kernel,
                        r_gate_kernel, e_bias, r_bias)
